# baseline (speedup 1.0000x reference)
from functools import lru_cache, partial

import numpy as np
import jax
import jax.numpy as jnp
from jax.experimental import pallas as pl
from jax.experimental.pallas import tpu as pltpu

_VMEM_LIMIT_BYTES = 58 * 1024 * 1024


def _nearest_indices(in_dim: int, out_dim: int) -> np.ndarray:
    src = np.floor(np.arange(out_dim, dtype=np.float32) * np.float32(in_dim / out_dim))
    return np.clip(src.astype(np.int64), 0, in_dim - 1)


@lru_cache(maxsize=16)
def _sel_w_mat(w_in: int, w_out: int):
    idx = _nearest_indices(w_in, w_out)
    m = np.zeros((w_in, w_out), dtype=np.float32)
    m[idx, np.arange(w_out)] = 1.0
    return jnp.asarray(m)


def _upsample_kernel(sel_w_ref, x_ref, o_ref, *, sf_h, n_split):
    h_in, w_in, c = x_ref.shape[1], x_ref.shape[2], x_ref.shape[3]
    cs = c // n_split
    j = pl.program_id(1)
    for jj in range(n_split):

        @pl.when(j == jj)
        def _():
            v = jnp.transpose(x_ref[0, :, :, jj * cs:(jj + 1) * cs], (2, 0, 1))
            v2 = v.reshape(cs * h_in, w_in)
            t = jnp.dot(v2, sel_w_ref[...], preferred_element_type=jnp.float32)
            for p in range(sf_h):
                o_ref[p::sf_h, :] = t


def kernel(x):
    N, C, H_in, W_in = x.shape
    sf_h = sf_w = 2
    H_out, W_out = H_in * sf_h, W_in * sf_w

    orig_dtype = x.dtype
    if not jnp.issubdtype(x.dtype, jnp.floating):
        x = x.astype(jnp.float32)

    sel_w = _sel_w_mat(W_in, W_out).astype(x.dtype)
    x_nhwc = jnp.transpose(x, (0, 2, 3, 1))

    n_split = 2 if C % 2 == 0 else 1
    out2d = pl.pallas_call(
        partial(_upsample_kernel, sf_h=sf_h, n_split=n_split),
        out_shape=jax.ShapeDtypeStruct((N * C * H_out, W_out), x.dtype),
        grid=(N, n_split),
        in_specs=[
            pl.BlockSpec((W_in, W_out), lambda n, j: (0, 0)),
            pl.BlockSpec((1, H_in, W_in, C), lambda n, j: (n, 0, 0, 0)),
        ],
        out_specs=pl.BlockSpec((C // n_split * H_out, W_out), lambda n, j: (n * n_split + j, 0)),
        compiler_params=pltpu.CompilerParams(
            dimension_semantics=("parallel", "arbitrary"),
            vmem_limit_bytes=_VMEM_LIMIT_BYTES,
        ),
    )(sel_w, x_nhwc)

    out = out2d.reshape(N, C, H_out, W_out)
    if out.dtype != orig_dtype:
        out = out.astype(orig_dtype)
    return out


# speedup vs baseline: 1.3169x; 1.3169x over previous
from functools import lru_cache, partial

import numpy as np
import jax
import jax.numpy as jnp
from jax.experimental import pallas as pl
from jax.experimental.pallas import tpu as pltpu

_VMEM_LIMIT_BYTES = 58 * 1024 * 1024


def _nearest_indices(in_dim: int, out_dim: int) -> np.ndarray:
    src = np.floor(np.arange(out_dim, dtype=np.float32) * np.float32(in_dim / out_dim))
    return np.clip(src.astype(np.int64), 0, in_dim - 1)


@lru_cache(maxsize=16)
def _sel_w_mat(w_in: int, w_out: int):
    idx = _nearest_indices(w_in, w_out)
    m = np.zeros((w_in, w_out), dtype=np.float32)
    m[idx, np.arange(w_out)] = 1.0
    return jnp.asarray(m)


def _upsample_kernel(sel_w_ref, x_ref, o_ref, *, sf_h):
    nb, h_in, w_in, c = x_ref.shape
    rows = c * h_in
    for b in range(nb):
        v = jnp.transpose(x_ref[b], (2, 0, 1))
        v2 = v.reshape(rows, w_in)
        t = jnp.dot(v2, sel_w_ref[...], preferred_element_type=jnp.float32)
        for j in range(sf_h):
            o_ref[b * sf_h * rows + j:(b + 1) * sf_h * rows:sf_h, :] = t


def kernel(x):
    N, C, H_in, W_in = x.shape
    sf_h = sf_w = 2
    H_out, W_out = H_in * sf_h, W_in * sf_w

    orig_dtype = x.dtype
    if not jnp.issubdtype(x.dtype, jnp.floating):
        x = x.astype(jnp.float32)

    sel_w = _sel_w_mat(W_in, W_out).astype(x.dtype)
    x_nhwc = jnp.transpose(x, (0, 2, 3, 1))

    nb = 2 if N % 2 == 0 else 1
    out2d = pl.pallas_call(
        partial(_upsample_kernel, sf_h=sf_h),
        out_shape=jax.ShapeDtypeStruct((N * C * H_out, W_out), x.dtype),
        grid=(N // nb,),
        in_specs=[
            pl.BlockSpec((W_in, W_out), lambda n: (0, 0)),
            pl.BlockSpec((nb, H_in, W_in, C), lambda n: (n, 0, 0, 0)),
        ],
        out_specs=pl.BlockSpec((nb * C * H_out, W_out), lambda n: (n, 0)),
        compiler_params=pltpu.CompilerParams(
            dimension_semantics=("parallel",),
            vmem_limit_bytes=_VMEM_LIMIT_BYTES,
        ),
    )(sel_w, x_nhwc)

    out = out2d.reshape(N, C, H_out, W_out)
    if out.dtype != orig_dtype:
        out = out.astype(orig_dtype)
    return out
